# baseline (speedup 1.0000x reference)
import functools

import numpy as np

import jax
import jax.numpy as jnp
from jax import lax
from jax.experimental import pallas as pl
from jax.experimental.pallas import tpu as pltpu
from jax.experimental.pallas import tpu_sc as plsc

_C = 1000
_CPAD = 1024
_D = 128
_N = 16384
_MOM = 0.9

_NC = 2
_NT = 16
_NW = _NC * _NT
_ROWS_PER_TILE = _N // _NW
_CHUNK = 256
_LCHUNK = 128
_NCHUNK = _ROWS_PER_TILE // _CHUNK
_LROWS = _ROWS_PER_TILE // _LCHUNK
_RB = _CPAD // _NT


def _sc_body(feat, lab, tfeat, tlab, zeros_hbm, zcnt_hbm, ones_hbm,
             sums_f, cnt_f, sums_t, cnt_t,
             fbuf0, fbuf1, fbuf2, lab_v, ones_v, zcnt_v,
             acc_f, acc_t, cntacc_f, cntacc_t,
             sem_init, sem_cnt, sem0, sem1, sem2):
    cid = lax.axis_index("c")
    sid = lax.axis_index("s")
    wid = cid * _NT + sid
    base = wid * _ROWS_PER_TILE

    init_f = [
        pltpu.async_copy(zeros_hbm, acc_f.at[pl.ds(sid * _RB, _RB)], sem_init),
        pltpu.async_copy(zcnt_hbm, zcnt_v, sem_init),
        pltpu.async_copy(lab.at[pl.ds(wid * _LROWS, _LROWS)],
                         lab_v.at[pl.ds(0, _LROWS)], sem_init),
        pltpu.async_copy(tlab.at[pl.ds(wid * _LROWS, _LROWS)],
                         lab_v.at[pl.ds(_LROWS, _LROWS)], sem_init),
        pltpu.async_copy(ones_hbm, ones_v, sem_init),
    ]

    bufs = (fbuf0, fbuf1, fbuf2)
    sems = (sem0, sem1, sem2)
    nchunks = 2 * _NCHUNK

    def chunk_src(i):
        src = feat if i < _NCHUNK else tfeat
        off = base + (i % _NCHUNK) * _CHUNK
        return src.at[pl.ds(off, _CHUNK)]

    copies = [pltpu.async_copy(chunk_src(i), bufs[i % 3], sems[i % 3])
              for i in range(min(3, nchunks))]

    for c in init_f:
        c.wait()
    pltpu.sync_copy(zcnt_v, cntacc_f.at[pl.ds(sid * _RB, _RB)])
    plsc.subcore_barrier()

    def scatter_chunks(lo, hi, acc, cntacc):
        cnt_copies = []
        for i in range(lo, hi):
            copies[i].wait()
            buf = bufs[i % 3]
            for h in range(_CHUNK // _LCHUNK):
                lrow = lab_v.at[i * (_CHUNK // _LCHUNK) + h]
                cnt_copies.append(
                    pltpu.async_copy(ones_v, cntacc.at[lrow], sem_cnt, add=True))
                pltpu.sync_copy(buf.at[pl.ds(h * _LCHUNK, _LCHUNK)],
                                acc.at[lrow], add=True)
            if i + 3 < nchunks:
                copies.append(
                    pltpu.async_copy(chunk_src(i + 3), bufs[(i + 3) % 3],
                                     sems[(i + 3) % 3]))
        for c in cnt_copies:
            c.wait()

    init_t = pltpu.async_copy(zeros_hbm, acc_t.at[pl.ds(sid * _RB, _RB)],
                              sem_init)
    scatter_chunks(0, _NCHUNK, acc_f, cntacc_f)
    init_t.wait()
    pltpu.sync_copy(zcnt_v, cntacc_t.at[pl.ds(sid * _RB, _RB)])
    plsc.subcore_barrier()

    out_f = pltpu.async_copy(acc_f.at[pl.ds(sid * _RB, _RB)],
                             sums_f.at[cid].at[pl.ds(sid * _RB, _RB)], sem_init)
    pltpu.sync_copy(cntacc_f.at[pl.ds(sid * _RB, _RB)], zcnt_v)
    pltpu.sync_copy(zcnt_v, cnt_f.at[cid].at[pl.ds(sid * _RB, _RB)])
    scatter_chunks(_NCHUNK, nchunks, acc_t, cntacc_t)
    plsc.subcore_barrier()

    out_t = pltpu.async_copy(acc_t.at[pl.ds(sid * _RB, _RB)],
                             sums_t.at[cid].at[pl.ds(sid * _RB, _RB)], sem_init)
    pltpu.sync_copy(cntacc_t.at[pl.ds(sid * _RB, _RB)], zcnt_v)
    pltpu.sync_copy(zcnt_v, cnt_t.at[cid].at[pl.ds(sid * _RB, _RB)])
    out_f.wait()
    out_t.wait()


_sc_segsum = functools.partial(
    pl.kernel,
    out_type=[
        jax.ShapeDtypeStruct((_NC, _CPAD, _D), jnp.float32),
        jax.ShapeDtypeStruct((_NC, _CPAD), jnp.float32),
        jax.ShapeDtypeStruct((_NC, _CPAD, _D), jnp.float32),
        jax.ShapeDtypeStruct((_NC, _CPAD), jnp.float32),
    ],
    mesh=plsc.VectorSubcoreMesh(core_axis_name="c", subcore_axis_name="s"),
    scratch_types=[
        pltpu.VMEM((_CHUNK, _D), jnp.float32),
        pltpu.VMEM((_CHUNK, _D), jnp.float32),
        pltpu.VMEM((_CHUNK, _D), jnp.float32),
        pltpu.VMEM((2 * _LROWS, _LCHUNK), jnp.int32),
        pltpu.VMEM((_LCHUNK,), jnp.float32),
        pltpu.VMEM((_RB,), jnp.float32),
        pltpu.VMEM_SHARED((_CPAD, _D), jnp.float32),
        pltpu.VMEM_SHARED((_CPAD, _D), jnp.float32),
        pltpu.VMEM_SHARED((_CPAD,), jnp.float32),
        pltpu.VMEM_SHARED((_CPAD,), jnp.float32),
        pltpu.SemaphoreType.DMA,
        pltpu.SemaphoreType.DMA,
        pltpu.SemaphoreType.DMA,
        pltpu.SemaphoreType.DMA,
        pltpu.SemaphoreType.DMA,
    ],
)(_sc_body)


_CB = 200


def _tc_body(sf_ref, cf_ref, st_ref, ct_ref, p_ref, tp_ref, dp_ref,
             newp_ref, newt_ref, loss_ref, acc_ref):
    i = pl.program_id(0)
    mom = jnp.float32(_MOM)
    one_minus = jnp.float32(1.0 - _MOM)
    lane_ones = jnp.ones((_NC, _D), jnp.float32)

    def upd(sums3, cnt2, proto):
        sums = sums3[0] + sums3[1]
        counts = lax.dot_general(cnt2, lane_ones, (((1,), (0,)), ((), ())),
                                 preferred_element_type=jnp.float32)
        means = sums / jnp.maximum(counts, 1.0)
        has = counts > 0.0
        uninit = jnp.sum(proto, axis=1, keepdims=True) == 0.0
        blended = jnp.where(uninit, means, mom * proto + one_minus * means)
        return jnp.where(has, blended, proto)

    newp = upd(sf_ref[...], cf_ref[...], p_ref[...])
    newt = upd(st_ref[...], ct_ref[...], tp_ref[...])
    newp_ref[...] = newp
    newt_ref[...] = newt

    dp = dp_ref[...]
    valid = (jnp.sum(newt, axis=1) != 0.0) & (jnp.sum(newp, axis=1) != 0.0)
    diff = newt - (newp + dp)
    align_err = jnp.sqrt(jnp.sum(diff * diff, axis=1))
    reg_pen = 0.5 * jnp.sqrt(jnp.sum(dp * dp, axis=1))
    per_class = jnp.where(valid, align_err + reg_pen, 0.0)
    psum = jnp.sum(per_class)
    nval = jnp.sum(valid.astype(jnp.float32))

    @pl.when(i == 0)
    def _():
        acc_ref[0] = psum
        acc_ref[1] = nval

    @pl.when(i > 0)
    def _():
        acc_ref[0] += psum
        acc_ref[1] += nval

    @pl.when(i == _C // _CB - 1)
    def _():
        tot, cnt = acc_ref[0], acc_ref[1]
        loss_ref[0, 0] = jnp.where(cnt > 0.0, tot / jnp.maximum(cnt, 1.0),
                                   jnp.float32(0.0))


_tc_finish = pl.pallas_call(
    _tc_body,
    grid=(_C // _CB,),
    in_specs=[
        pl.BlockSpec((_NC, _CB, _D), lambda i: (0, i, 0)),
        pl.BlockSpec((_CB, _NC), lambda i: (i, 0)),
        pl.BlockSpec((_NC, _CB, _D), lambda i: (0, i, 0)),
        pl.BlockSpec((_CB, _NC), lambda i: (i, 0)),
        pl.BlockSpec((_CB, _D), lambda i: (i, 0)),
        pl.BlockSpec((_CB, _D), lambda i: (i, 0)),
        pl.BlockSpec((_CB, _D), lambda i: (i, 0)),
    ],
    out_specs=[
        pl.BlockSpec((_CB, _D), lambda i: (i, 0)),
        pl.BlockSpec((_CB, _D), lambda i: (i, 0)),
        pl.BlockSpec(memory_space=pltpu.SMEM),
    ],
    scratch_shapes=[pltpu.SMEM((2,), jnp.float32)],
    out_shape=[
        jax.ShapeDtypeStruct((_C, _D), jnp.float32),
        jax.ShapeDtypeStruct((_C, _D), jnp.float32),
        jax.ShapeDtypeStruct((1, 1), jnp.float32),
    ],
)


def kernel(features, labels, t_features, t_pseudo_labels, prototypes,
           target_prototypes, delta_phi):
    lab2d = labels.astype(jnp.int32).reshape(_N // _LCHUNK, _LCHUNK)
    tlab2d = t_pseudo_labels.astype(jnp.int32).reshape(_N // _LCHUNK, _LCHUNK)
    zeros2d = np.zeros((_RB, _D), np.float32)
    zcnt1d = np.zeros((_RB,), np.float32)
    ones1d = np.ones((_LCHUNK,), np.float32)
    sums_f, cnt_f, sums_t, cnt_t = _sc_segsum(
        features, lab2d, t_features, tlab2d, zeros2d, zcnt1d, ones1d)
    newp, newt, loss = _tc_finish(
        sums_f, cnt_f.T, sums_t, cnt_t.T,
        prototypes, target_prototypes, delta_phi)
    return newp, newt, loss[0, 0]


# speedup vs baseline: 1.1335x; 1.1335x over previous
import functools

import numpy as np

import jax
import jax.numpy as jnp
from jax import lax
from jax.experimental import pallas as pl
from jax.experimental.pallas import tpu as pltpu
from jax.experimental.pallas import tpu_sc as plsc

_C = 1000
_CPAD = 1024
_D = 128
_N = 16384
_MOM = 0.9

_NC = 2
_NT = 16
_NW = _NC * _NT
_ROWS_PER_TILE = _N // _NW
_CHUNK = 256
_LCHUNK = 128
_NCHUNK = _ROWS_PER_TILE // _CHUNK
_LROWS = _ROWS_PER_TILE // _LCHUNK
_RB = _CPAD // _NT


def _sc_body(feat, lab, tfeat, tlab, zeros_hbm, zcnt_hbm, ones_hbm,
             sums_f, cnt_f, sums_t, cnt_t,
             fbuf0, fbuf1, fbuf2, lab_v, ones_v, zcnt_v,
             acc_f, acc_t, cntacc_f, cntacc_t,
             sem_init, sem_cnt, sem0, sem1, sem2):
    cid = lax.axis_index("c")
    sid = lax.axis_index("s")
    wid = cid * _NT + sid
    base = wid * _ROWS_PER_TILE

    init_f = [
        pltpu.async_copy(zeros_hbm, acc_f.at[pl.ds(sid * _RB, _RB)], sem_init),
        pltpu.async_copy(zcnt_hbm, zcnt_v, sem_init),
        pltpu.async_copy(lab.at[pl.ds(wid * _LROWS, _LROWS)],
                         lab_v.at[pl.ds(0, _LROWS)], sem_init),
        pltpu.async_copy(tlab.at[pl.ds(wid * _LROWS, _LROWS)],
                         lab_v.at[pl.ds(_LROWS, _LROWS)], sem_init),
        pltpu.async_copy(ones_hbm, ones_v, sem_init),
    ]

    bufs = (fbuf0, fbuf1, fbuf2)
    sems = (sem0, sem1, sem2)
    nchunks = 2 * _NCHUNK

    def chunk_src(i):
        src = feat if i < _NCHUNK else tfeat
        off = base + (i % _NCHUNK) * _CHUNK
        return src.at[pl.ds(off, _CHUNK)]

    copies = [pltpu.async_copy(chunk_src(i), bufs[i % 3], sems[i % 3])
              for i in range(min(3, nchunks))]

    for c in init_f:
        c.wait()
    pltpu.sync_copy(zcnt_v, cntacc_f.at[pl.ds(sid * _RB, _RB)])
    plsc.subcore_barrier()

    def scatter_chunks(lo, hi, acc, cntacc):
        cnt_copies = []
        for i in range(lo, hi):
            copies[i].wait()
            buf = bufs[i % 3]
            for h in range(_CHUNK // _LCHUNK):
                lrow = lab_v.at[i * (_CHUNK // _LCHUNK) + h]
                cnt_copies.append(
                    pltpu.async_copy(ones_v, cntacc.at[lrow], sem_cnt, add=True))
                pltpu.sync_copy(buf.at[pl.ds(h * _LCHUNK, _LCHUNK)],
                                acc.at[lrow], add=True)
            if i + 3 < nchunks:
                copies.append(
                    pltpu.async_copy(chunk_src(i + 3), bufs[(i + 3) % 3],
                                     sems[(i + 3) % 3]))
        for c in cnt_copies:
            c.wait()

    init_t = pltpu.async_copy(zeros_hbm, acc_t.at[pl.ds(sid * _RB, _RB)],
                              sem_init)
    scatter_chunks(0, _NCHUNK, acc_f, cntacc_f)
    init_t.wait()
    pltpu.sync_copy(zcnt_v, cntacc_t.at[pl.ds(sid * _RB, _RB)])
    plsc.subcore_barrier()

    out_f = pltpu.async_copy(acc_f.at[pl.ds(sid * _RB, _RB)],
                             sums_f.at[cid].at[pl.ds(sid * _RB, _RB)], sem_init)
    pltpu.sync_copy(cntacc_f.at[pl.ds(sid * _RB, _RB)], zcnt_v)
    pltpu.sync_copy(zcnt_v, cnt_f.at[cid].at[pl.ds(sid * _RB, _RB)])
    scatter_chunks(_NCHUNK, nchunks, acc_t, cntacc_t)
    plsc.subcore_barrier()

    out_t = pltpu.async_copy(acc_t.at[pl.ds(sid * _RB, _RB)],
                             sums_t.at[cid].at[pl.ds(sid * _RB, _RB)], sem_init)
    pltpu.sync_copy(cntacc_t.at[pl.ds(sid * _RB, _RB)], zcnt_v)
    pltpu.sync_copy(zcnt_v, cnt_t.at[cid].at[pl.ds(sid * _RB, _RB)])
    out_f.wait()
    out_t.wait()


_sc_segsum = functools.partial(
    pl.kernel,
    out_type=[
        jax.ShapeDtypeStruct((_NC, _CPAD, _D), jnp.float32),
        jax.ShapeDtypeStruct((_NC, _CPAD), jnp.float32),
        jax.ShapeDtypeStruct((_NC, _CPAD, _D), jnp.float32),
        jax.ShapeDtypeStruct((_NC, _CPAD), jnp.float32),
    ],
    mesh=plsc.VectorSubcoreMesh(core_axis_name="c", subcore_axis_name="s"),
    scratch_types=[
        pltpu.VMEM((_CHUNK, _D), jnp.float32),
        pltpu.VMEM((_CHUNK, _D), jnp.float32),
        pltpu.VMEM((_CHUNK, _D), jnp.float32),
        pltpu.VMEM((2 * _LROWS, _LCHUNK), jnp.int32),
        pltpu.VMEM((_LCHUNK,), jnp.float32),
        pltpu.VMEM((_RB,), jnp.float32),
        pltpu.VMEM_SHARED((_CPAD, _D), jnp.float32),
        pltpu.VMEM_SHARED((_CPAD, _D), jnp.float32),
        pltpu.VMEM_SHARED((_CPAD,), jnp.float32),
        pltpu.VMEM_SHARED((_CPAD,), jnp.float32),
        pltpu.SemaphoreType.DMA,
        pltpu.SemaphoreType.DMA,
        pltpu.SemaphoreType.DMA,
        pltpu.SemaphoreType.DMA,
        pltpu.SemaphoreType.DMA,
    ],
)(_sc_body)


def _tc_body(sf_ref, cf_ref, st_ref, ct_ref, p_ref, tp_ref, dp_ref,
             newp_ref, newt_ref, loss_ref):
    mom = jnp.float32(_MOM)
    one_minus = jnp.float32(1.0 - _MOM)
    lane_ones = jnp.ones((_NC, _D), jnp.float32)

    def upd(sums3, cnt2, proto):
        sums = (sums3[0] + sums3[1])[:_C]
        counts = lax.dot_general(cnt2, lane_ones, (((0,), (0,)), ((), ())),
                                 preferred_element_type=jnp.float32)[:_C]
        means = sums / jnp.maximum(counts, 1.0)
        has = counts > 0.0
        uninit = jnp.sum(proto, axis=1, keepdims=True) == 0.0
        blended = jnp.where(uninit, means, mom * proto + one_minus * means)
        return jnp.where(has, blended, proto)

    newp = upd(sf_ref[...], cf_ref[...], p_ref[...])
    newt = upd(st_ref[...], ct_ref[...], tp_ref[...])
    newp_ref[...] = newp
    newt_ref[...] = newt

    dp = dp_ref[...]
    valid = (jnp.sum(newt, axis=1) != 0.0) & (jnp.sum(newp, axis=1) != 0.0)
    diff = newt - (newp + dp)
    align_err = jnp.sqrt(jnp.sum(diff * diff, axis=1))
    reg_pen = 0.5 * jnp.sqrt(jnp.sum(dp * dp, axis=1))
    per_class = jnp.where(valid, align_err + reg_pen, 0.0)
    nvalid = jnp.sum(valid.astype(jnp.float32))
    loss = jnp.where(nvalid > 0.0, jnp.sum(per_class) / jnp.maximum(nvalid, 1.0),
                     jnp.float32(0.0))
    loss_ref[...] = jnp.broadcast_to(loss, (1, 1))


_tc_finish = pl.pallas_call(
    _tc_body,
    out_shape=[
        jax.ShapeDtypeStruct((_C, _D), jnp.float32),
        jax.ShapeDtypeStruct((_C, _D), jnp.float32),
        jax.ShapeDtypeStruct((1, 1), jnp.float32),
    ],
)


def kernel(features, labels, t_features, t_pseudo_labels, prototypes,
           target_prototypes, delta_phi):
    lab2d = labels.astype(jnp.int32).reshape(_N // _LCHUNK, _LCHUNK)
    tlab2d = t_pseudo_labels.astype(jnp.int32).reshape(_N // _LCHUNK, _LCHUNK)
    zeros2d = np.zeros((_RB, _D), np.float32)
    zcnt1d = np.zeros((_RB,), np.float32)
    ones1d = np.ones((_LCHUNK,), np.float32)
    sums_f, cnt_f, sums_t, cnt_t = _sc_segsum(
        features, lab2d, t_features, tlab2d, zeros2d, zcnt1d, ones1d)
    newp, newt, loss = _tc_finish(
        sums_f, cnt_f, sums_t, cnt_t,
        prototypes, target_prototypes, delta_phi)
    return newp, newt, loss[0, 0]
